# Initial kernel scaffold; baseline (speedup 1.0000x reference)
import jax
import jax.numpy as jnp
from jax.experimental import pallas as pl


def kernel(x, edge_index, W1, b1, W2, b2, W3, b3):
    raise NotImplementedError("write your pallas kernel here")


# speedup vs baseline: 17.8819x; 17.8819x over previous
import functools

import jax
import jax.numpy as jnp
from jax import lax
from jax.experimental import pallas as pl
from jax.experimental.pallas import tpu as pltpu
from jax.experimental.pallas import tpu_sc as plsc

N = 10000
E = 320000
D_IN = 128
NC, NS = 2, 16
NW = NC * NS
CHUNK = 128
CPT = 80
E_PAD = NW * CPT * CHUNK
N_PAD = 10240
RPT = N_PAD // NS
DEGW = 128

ROW_BLK = 400
N_BLKS = N // ROW_BLK

@functools.cache
def _sc_mesh():
    return plsc.VectorSubcoreMesh(
        core_axis_name="c", subcore_axis_name="s", num_cores=NC, num_subcores=NS
    )


def _sc_degree(dst_rows, ones_col, zero_col):
    @functools.partial(
        pl.kernel,
        out_type=jax.ShapeDtypeStruct((NC, N_PAD, DEGW), jnp.float32),
        mesh=_sc_mesh(),
        scratch_types=[
            pltpu.VMEM((CPT, CHUNK), jnp.int32),
            pltpu.VMEM((CHUNK, DEGW), jnp.float32),
            pltpu.VMEM_SHARED((N_PAD, DEGW), jnp.float32),
        ],
    )
    def deg_kernel(dst_hbm, ones_hbm, zcol_hbm, out_hbm, dst_v, ones_v, acc):
        c = lax.axis_index("c")
        s = lax.axis_index("s")
        w = c * NS + s
        pltpu.sync_copy(zcol_hbm, acc.at[pl.ds(s * RPT, RPT)])
        pltpu.sync_copy(ones_hbm, ones_v)
        pltpu.sync_copy(dst_hbm.at[pl.ds(w * CPT, CPT)], dst_v)
        plsc.subcore_barrier()

        def body(j, carry):
            pltpu.sync_copy(ones_v, acc.at[dst_v.at[j]], add=True)
            return carry

        lax.fori_loop(0, CPT, body, 0)
        plsc.subcore_barrier()
        pltpu.sync_copy(acc.at[pl.ds(s * RPT, RPT)],
                        out_hbm.at[c, pl.ds(s * RPT, RPT)])

    return deg_kernel(dst_rows, ones_col, zero_col)


def _sc_propagate(hs, src_rows, dst_rows, zero_rows):
    @functools.partial(
        pl.kernel,
        out_type=jax.ShapeDtypeStruct((NC, N_PAD, D_IN), jnp.float32),
        mesh=_sc_mesh(),
        scratch_types=[
            pltpu.VMEM((CPT, CHUNK), jnp.int32),
            pltpu.VMEM((CPT, CHUNK), jnp.int32),
            pltpu.VMEM((CHUNK, D_IN), jnp.float32),
            pltpu.VMEM_SHARED((N_PAD, D_IN), jnp.float32),
            pltpu.SemaphoreType.DMA,
        ],
    )
    def prop_kernel(hs_hbm, src_hbm, dst_hbm, zr_hbm, out_hbm,
                    src_v, dst_v, rows_v, acc, sem):
        c = lax.axis_index("c")
        s = lax.axis_index("s")
        w = c * NS + s
        pltpu.sync_copy(zr_hbm, acc.at[pl.ds(s * RPT, RPT)])
        pltpu.sync_copy(src_hbm.at[pl.ds(w * CPT, CPT)], src_v)
        pltpu.sync_copy(dst_hbm.at[pl.ds(w * CPT, CPT)], dst_v)
        plsc.subcore_barrier()

        def body(j, carry):
            pltpu.async_copy(hs_hbm.at[src_v.at[j]], rows_v, sem).wait()
            pltpu.sync_copy(rows_v, acc.at[dst_v.at[j]], add=True)
            return carry

        lax.fori_loop(0, CPT, body, 0)
        plsc.subcore_barrier()
        pltpu.sync_copy(acc.at[pl.ds(s * RPT, RPT)],
                        out_hbm.at[c, pl.ds(s * RPT, RPT)])

    return prop_kernel(hs, src_rows, dst_rows, zero_rows)


def _tc_prescale(degp, x):
    def body(dp_ref, x_ref, dinv_ref, xs_ref):
        deg = dp_ref[0][:, :1] + dp_ref[1][:, :1] + 1.0
        dinv = lax.rsqrt(deg)
        dinv_ref[...] = dinv
        xs_ref[...] = x_ref[...] * dinv

    return pl.pallas_call(
        body,
        grid=(N_BLKS,),
        in_specs=[
            pl.BlockSpec((NC, ROW_BLK, DEGW), lambda i: (0, i, 0)),
            pl.BlockSpec((ROW_BLK, D_IN), lambda i: (i, 0)),
        ],
        out_specs=[
            pl.BlockSpec((ROW_BLK, 1), lambda i: (i, 0)),
            pl.BlockSpec((ROW_BLK, D_IN), lambda i: (i, 0)),
        ],
        out_shape=[
            jax.ShapeDtypeStruct((N, 1), jnp.float32),
            jax.ShapeDtypeStruct((N, D_IN), jnp.float32),
        ],
    )(degp, x)


def _tc_layer1(p0, xs, dinv, W1, b1, W2):
    def body(p_ref, xs_ref, dinv_ref, W1_ref, b1_ref, W2_ref, oa_ref, ob_ref):
        dinv = dinv_ref[...]
        p = (p_ref[0] + p_ref[1] + xs_ref[...]) * dinv
        z1 = jnp.maximum(
            jnp.dot(p, W1_ref[...], preferred_element_type=jnp.float32)
            + b1_ref[...], 0.0)
        h2 = jnp.dot(z1, W2_ref[...], preferred_element_type=jnp.float32)
        oa_ref[...] = h2[:, :128] * dinv
        ob_ref[...] = h2[:, 128:] * dinv

    return pl.pallas_call(
        body,
        grid=(N_BLKS,),
        in_specs=[
            pl.BlockSpec((NC, ROW_BLK, D_IN), lambda i: (0, i, 0)),
            pl.BlockSpec((ROW_BLK, D_IN), lambda i: (i, 0)),
            pl.BlockSpec((ROW_BLK, 1), lambda i: (i, 0)),
            pl.BlockSpec((D_IN, 512), lambda i: (0, 0)),
            pl.BlockSpec((1, 512), lambda i: (0, 0)),
            pl.BlockSpec((512, 256), lambda i: (0, 0)),
        ],
        out_specs=[
            pl.BlockSpec((ROW_BLK, 128), lambda i: (i, 0)),
            pl.BlockSpec((ROW_BLK, 128), lambda i: (i, 0)),
        ],
        out_shape=[
            jax.ShapeDtypeStruct((N, 128), jnp.float32),
            jax.ShapeDtypeStruct((N, 128), jnp.float32),
        ],
    )(p0, xs, dinv, W1, b1, W2)


def _tc_layer2(p2a, p2b, hs2a, hs2b, dinv, b2, W3):
    def body(pa_ref, pb_ref, ha_ref, hb_ref, dinv_ref, b2_ref, W3_ref, o_ref):
        dinv = dinv_ref[...]
        za = jnp.maximum(
            (pa_ref[0] + pa_ref[1] + ha_ref[...]) * dinv + b2_ref[:, :128], 0.0)
        zb = jnp.maximum(
            (pb_ref[0] + pb_ref[1] + hb_ref[...]) * dinv + b2_ref[:, 128:], 0.0)
        h3 = (jnp.dot(za, W3_ref[:128, :], preferred_element_type=jnp.float32)
              + jnp.dot(zb, W3_ref[128:, :], preferred_element_type=jnp.float32))
        o_ref[...] = h3 * dinv

    return pl.pallas_call(
        body,
        grid=(N_BLKS,),
        in_specs=[
            pl.BlockSpec((NC, ROW_BLK, 128), lambda i: (0, i, 0)),
            pl.BlockSpec((NC, ROW_BLK, 128), lambda i: (0, i, 0)),
            pl.BlockSpec((ROW_BLK, 128), lambda i: (i, 0)),
            pl.BlockSpec((ROW_BLK, 128), lambda i: (i, 0)),
            pl.BlockSpec((ROW_BLK, 1), lambda i: (i, 0)),
            pl.BlockSpec((1, 256), lambda i: (0, 0)),
            pl.BlockSpec((256, 128), lambda i: (0, 0)),
        ],
        out_specs=pl.BlockSpec((ROW_BLK, 128), lambda i: (i, 0)),
        out_shape=jax.ShapeDtypeStruct((N, 128), jnp.float32),
    )(p2a, p2b, hs2a, hs2b, dinv, b2, W3)


def _tc_layer3(p3, hs3, dinv, b3):
    def body(p_ref, h_ref, dinv_ref, b3_ref, z_ref, s_ref):
        z = jnp.maximum(
            (p_ref[0] + p_ref[1] + h_ref[...]) * dinv_ref[...] + b3_ref[...],
            0.0)
        z_ref[...] = z
        prev = jnp.where(pl.program_id(0) == 0, 0.0, s_ref[0, 0])
        s_ref[0, 0] = prev + jnp.sum(z)

    return pl.pallas_call(
        body,
        grid=(N_BLKS,),
        in_specs=[
            pl.BlockSpec((NC, ROW_BLK, 128), lambda i: (0, i, 0)),
            pl.BlockSpec((ROW_BLK, 128), lambda i: (i, 0)),
            pl.BlockSpec((ROW_BLK, 1), lambda i: (i, 0)),
            pl.BlockSpec((1, 128), lambda i: (0, 0)),
        ],
        out_specs=[
            pl.BlockSpec((ROW_BLK, 128), lambda i: (i, 0)),
            pl.BlockSpec(memory_space=pltpu.SMEM),
        ],
        out_shape=[
            jax.ShapeDtypeStruct((N, 128), jnp.float32),
            jax.ShapeDtypeStruct((1, 1), jnp.float32),
        ],
    )(p3, hs3, dinv, b3)


def _tc_tail(z3, S):
    def body(z_ref, s_ref, o_ref):
        z = z_ref[...] / s_ref[0, 0]
        t = jnp.tanh(z)
        t = t * t
        rn = jnp.sqrt(jnp.sum(t * t, axis=1, keepdims=True))
        o_ref[...] = t / jnp.maximum(rn, 1e-12)

    return pl.pallas_call(
        body,
        grid=(N_BLKS,),
        in_specs=[
            pl.BlockSpec((ROW_BLK, 128), lambda i: (i, 0)),
            pl.BlockSpec(memory_space=pltpu.SMEM),
        ],
        out_specs=pl.BlockSpec((ROW_BLK, 128), lambda i: (i, 0)),
        out_shape=jax.ShapeDtypeStruct((N, 128), jnp.float32),
    )(z3, S)


def kernel(x, edge_index, W1, b1, W2, b2, W3, b3):
    src = edge_index[0].astype(jnp.int32)
    dst = edge_index[1].astype(jnp.int32)
    padn = E_PAD - E
    ar = jnp.arange(padn, dtype=jnp.int32)
    src_rows = jnp.concatenate([src, ar % N]).reshape(E_PAD // CHUNK, CHUNK)
    dst_rows = jnp.concatenate([dst, N + ar % (N_PAD - N)]).reshape(
        E_PAD // CHUNK, CHUNK)
    ones_col = jnp.ones((CHUNK, DEGW), jnp.float32)
    zero_col = jnp.zeros((RPT, DEGW), jnp.float32)
    zero_rows = jnp.zeros((RPT, D_IN), jnp.float32)

    degp = _sc_degree(dst_rows, ones_col, zero_col)
    dinv, xs = _tc_prescale(degp, x)
    p0 = _sc_propagate(xs, src_rows, dst_rows, zero_rows)
    hs2a, hs2b = _tc_layer1(p0, xs, dinv, W1, jnp.reshape(b1, (1, 512)), W2)
    p2a = _sc_propagate(hs2a, src_rows, dst_rows, zero_rows)
    p2b = _sc_propagate(hs2b, src_rows, dst_rows, zero_rows)
    hs3 = _tc_layer2(p2a, p2b, hs2a, hs2b, dinv, jnp.reshape(b2, (1, 256)), W3)
    p3 = _sc_propagate(hs3, src_rows, dst_rows, zero_rows)
    z3, S = _tc_layer3(p3, hs3, dinv, jnp.reshape(b3, (1, 128)))
    return _tc_tail(z3, S)


# speedup vs baseline: 19.8286x; 1.1089x over previous
import functools

import jax
import jax.numpy as jnp
from jax import lax
from jax.experimental import pallas as pl
from jax.experimental.pallas import tpu as pltpu
from jax.experimental.pallas import tpu_sc as plsc

N = 10000
E = 320000
D_IN = 128
NC, NS = 2, 16
NW = NC * NS
CHUNK = 128
CPT = 80
E_PAD = NW * CPT * CHUNK
N_PAD = 10240
RPT = N_PAD // NS
DEGW = 128

NBUF = 5
PBUF = 2
IB = 16
N_IBLK = CPT // IB
P_INNER = IB // PBUF

ROW_BLK = 400
N_BLKS = N // ROW_BLK

@functools.cache
def _sc_mesh():
    return plsc.VectorSubcoreMesh(
        core_axis_name="c", subcore_axis_name="s", num_cores=NC, num_subcores=NS
    )


def _sc_degree(dst_rows, ones_col, zero_col):
    @functools.partial(
        pl.kernel,
        out_type=jax.ShapeDtypeStruct((NC, N_PAD, DEGW), jnp.float32),
        mesh=_sc_mesh(),
        scratch_types=[
            pltpu.VMEM((CPT, CHUNK), jnp.int32),
            pltpu.VMEM((CHUNK, DEGW), jnp.float32),
            pltpu.VMEM_SHARED((N_PAD, DEGW), jnp.float32),
        ] + [pltpu.SemaphoreType.DMA] * NBUF,
    )
    def deg_kernel(dst_hbm, ones_hbm, zcol_hbm, out_hbm, dst_v, ones_v, acc,
                   *sems):
        c = lax.axis_index("c")
        s = lax.axis_index("s")
        w = c * NS + s
        pltpu.sync_copy(zcol_hbm, acc.at[pl.ds(s * RPT, RPT)])
        pltpu.sync_copy(ones_hbm, ones_v)
        pltpu.sync_copy(dst_hbm.at[pl.ds(w * CPT, CPT)], dst_v)
        plsc.subcore_barrier()

        def body(t, carry):
            base = t * NBUF
            descs = [
                pltpu.async_copy(ones_v, acc.at[dst_v.at[base + b]],
                                 sems[b], add=True)
                for b in range(NBUF)
            ]
            for d in descs:
                d.wait()
            return carry

        lax.fori_loop(0, CPT // NBUF, body, 0)
        plsc.subcore_barrier()
        pltpu.sync_copy(acc.at[pl.ds(s * RPT, RPT)],
                        out_hbm.at[c, pl.ds(s * RPT, RPT)])

    return deg_kernel(dst_rows, ones_col, zero_col)


def _sc_propagate(hs, src_rows, dst_rows, zero_rows):
    @functools.partial(
        pl.kernel,
        out_type=jax.ShapeDtypeStruct((NC, N_PAD, D_IN), jnp.float32),
        mesh=_sc_mesh(),
        scratch_types=[
            pltpu.VMEM((IB, CHUNK), jnp.int32),
            pltpu.VMEM((IB, CHUNK), jnp.int32),
        ] + [pltpu.VMEM((CHUNK, D_IN), jnp.float32)] * PBUF + [
            pltpu.VMEM_SHARED((N_PAD, D_IN), jnp.float32),
        ] + [pltpu.SemaphoreType.DMA] * (2 * PBUF),
    )
    def prop_kernel(hs_hbm, src_hbm, dst_hbm, zr_hbm, out_hbm,
                    src_v, dst_v, *rest):
        bufs = rest[:PBUF]
        acc = rest[PBUF]
        gsems = rest[PBUF + 1:PBUF + 1 + PBUF]
        ssems = rest[PBUF + 1 + PBUF:]
        c = lax.axis_index("c")
        s = lax.axis_index("s")
        w = c * NS + s
        pltpu.sync_copy(zr_hbm, acc.at[pl.ds(s * RPT, RPT)])
        plsc.subcore_barrier()

        def _drain_scatter(b):
            pltpu.make_async_copy(bufs[b], acc.at[dst_v.at[0]],
                                  ssems[b]).wait()

        def block(ib, carry):
            @pl.when(ib > 0)
            def _():
                for b in range(PBUF):
                    _drain_scatter(b)

            boff = w * CPT + ib * IB
            pltpu.sync_copy(src_hbm.at[pl.ds(boff, IB)], src_v)
            pltpu.sync_copy(dst_hbm.at[pl.ds(boff, IB)], dst_v)

            def body(t, carry2):
                base = t * PBUF

                @pl.when(t > 0)
                def _():
                    for b in range(PBUF):
                        _drain_scatter(b)

                gds = [
                    pltpu.async_copy(hs_hbm.at[src_v.at[base + b]], bufs[b],
                                     gsems[b])
                    for b in range(PBUF)
                ]
                for b in range(PBUF):
                    gds[b].wait()
                    pltpu.async_copy(bufs[b], acc.at[dst_v.at[base + b]],
                                     ssems[b], add=True)
                return carry2

            lax.fori_loop(0, P_INNER, body, 0)
            return carry

        lax.fori_loop(0, N_IBLK, block, 0)
        for b in range(PBUF):
            _drain_scatter(b)
        plsc.subcore_barrier()
        pltpu.sync_copy(acc.at[pl.ds(s * RPT, RPT)],
                        out_hbm.at[c, pl.ds(s * RPT, RPT)])

    return prop_kernel(hs, src_rows, dst_rows, zero_rows)


def _tc_prescale(degp, x):
    def body(dp_ref, x_ref, dinv_ref, xs_ref):
        deg = dp_ref[0][:, :1] + dp_ref[1][:, :1] + 1.0
        dinv = lax.rsqrt(deg)
        dinv_ref[...] = dinv
        xs_ref[...] = x_ref[...] * dinv

    return pl.pallas_call(
        body,
        grid=(N_BLKS,),
        in_specs=[
            pl.BlockSpec((NC, ROW_BLK, DEGW), lambda i: (0, i, 0)),
            pl.BlockSpec((ROW_BLK, D_IN), lambda i: (i, 0)),
        ],
        out_specs=[
            pl.BlockSpec((ROW_BLK, 1), lambda i: (i, 0)),
            pl.BlockSpec((ROW_BLK, D_IN), lambda i: (i, 0)),
        ],
        out_shape=[
            jax.ShapeDtypeStruct((N, 1), jnp.float32),
            jax.ShapeDtypeStruct((N, D_IN), jnp.float32),
        ],
    )(degp, x)


def _tc_layer1(p0, xs, dinv, W1, b1, W2):
    def body(p_ref, xs_ref, dinv_ref, W1_ref, b1_ref, W2_ref, oa_ref, ob_ref):
        dinv = dinv_ref[...]
        p = (p_ref[0] + p_ref[1] + xs_ref[...]) * dinv
        z1 = jnp.maximum(
            jnp.dot(p, W1_ref[...], preferred_element_type=jnp.float32)
            + b1_ref[...], 0.0)
        h2 = jnp.dot(z1, W2_ref[...], preferred_element_type=jnp.float32)
        oa_ref[...] = h2[:, :128] * dinv
        ob_ref[...] = h2[:, 128:] * dinv

    return pl.pallas_call(
        body,
        grid=(N_BLKS,),
        in_specs=[
            pl.BlockSpec((NC, ROW_BLK, D_IN), lambda i: (0, i, 0)),
            pl.BlockSpec((ROW_BLK, D_IN), lambda i: (i, 0)),
            pl.BlockSpec((ROW_BLK, 1), lambda i: (i, 0)),
            pl.BlockSpec((D_IN, 512), lambda i: (0, 0)),
            pl.BlockSpec((1, 512), lambda i: (0, 0)),
            pl.BlockSpec((512, 256), lambda i: (0, 0)),
        ],
        out_specs=[
            pl.BlockSpec((ROW_BLK, 128), lambda i: (i, 0)),
            pl.BlockSpec((ROW_BLK, 128), lambda i: (i, 0)),
        ],
        out_shape=[
            jax.ShapeDtypeStruct((N, 128), jnp.float32),
            jax.ShapeDtypeStruct((N, 128), jnp.float32),
        ],
    )(p0, xs, dinv, W1, b1, W2)


def _tc_layer2(p2a, p2b, hs2a, hs2b, dinv, b2, W3):
    def body(pa_ref, pb_ref, ha_ref, hb_ref, dinv_ref, b2_ref, W3_ref, o_ref):
        dinv = dinv_ref[...]
        za = jnp.maximum(
            (pa_ref[0] + pa_ref[1] + ha_ref[...]) * dinv + b2_ref[:, :128], 0.0)
        zb = jnp.maximum(
            (pb_ref[0] + pb_ref[1] + hb_ref[...]) * dinv + b2_ref[:, 128:], 0.0)
        h3 = (jnp.dot(za, W3_ref[:128, :], preferred_element_type=jnp.float32)
              + jnp.dot(zb, W3_ref[128:, :], preferred_element_type=jnp.float32))
        o_ref[...] = h3 * dinv

    return pl.pallas_call(
        body,
        grid=(N_BLKS,),
        in_specs=[
            pl.BlockSpec((NC, ROW_BLK, 128), lambda i: (0, i, 0)),
            pl.BlockSpec((NC, ROW_BLK, 128), lambda i: (0, i, 0)),
            pl.BlockSpec((ROW_BLK, 128), lambda i: (i, 0)),
            pl.BlockSpec((ROW_BLK, 128), lambda i: (i, 0)),
            pl.BlockSpec((ROW_BLK, 1), lambda i: (i, 0)),
            pl.BlockSpec((1, 256), lambda i: (0, 0)),
            pl.BlockSpec((256, 128), lambda i: (0, 0)),
        ],
        out_specs=pl.BlockSpec((ROW_BLK, 128), lambda i: (i, 0)),
        out_shape=jax.ShapeDtypeStruct((N, 128), jnp.float32),
    )(p2a, p2b, hs2a, hs2b, dinv, b2, W3)


def _tc_layer3(p3, hs3, dinv, b3):
    def body(p_ref, h_ref, dinv_ref, b3_ref, z_ref, s_ref):
        z = jnp.maximum(
            (p_ref[0] + p_ref[1] + h_ref[...]) * dinv_ref[...] + b3_ref[...],
            0.0)
        z_ref[...] = z
        prev = jnp.where(pl.program_id(0) == 0, 0.0, s_ref[0, 0])
        s_ref[0, 0] = prev + jnp.sum(z)

    return pl.pallas_call(
        body,
        grid=(N_BLKS,),
        in_specs=[
            pl.BlockSpec((NC, ROW_BLK, 128), lambda i: (0, i, 0)),
            pl.BlockSpec((ROW_BLK, 128), lambda i: (i, 0)),
            pl.BlockSpec((ROW_BLK, 1), lambda i: (i, 0)),
            pl.BlockSpec((1, 128), lambda i: (0, 0)),
        ],
        out_specs=[
            pl.BlockSpec((ROW_BLK, 128), lambda i: (i, 0)),
            pl.BlockSpec(memory_space=pltpu.SMEM),
        ],
        out_shape=[
            jax.ShapeDtypeStruct((N, 128), jnp.float32),
            jax.ShapeDtypeStruct((1, 1), jnp.float32),
        ],
    )(p3, hs3, dinv, b3)


def _tc_tail(z3, S):
    def body(z_ref, s_ref, o_ref):
        z = z_ref[...] / s_ref[0, 0]
        t = jnp.tanh(z)
        t = t * t
        rn = jnp.sqrt(jnp.sum(t * t, axis=1, keepdims=True))
        o_ref[...] = t / jnp.maximum(rn, 1e-12)

    return pl.pallas_call(
        body,
        grid=(N_BLKS,),
        in_specs=[
            pl.BlockSpec((ROW_BLK, 128), lambda i: (i, 0)),
            pl.BlockSpec(memory_space=pltpu.SMEM),
        ],
        out_specs=pl.BlockSpec((ROW_BLK, 128), lambda i: (i, 0)),
        out_shape=jax.ShapeDtypeStruct((N, 128), jnp.float32),
    )(z3, S)


def kernel(x, edge_index, W1, b1, W2, b2, W3, b3):
    src = edge_index[0].astype(jnp.int32)
    dst = edge_index[1].astype(jnp.int32)
    padn = E_PAD - E
    ar = jnp.arange(padn, dtype=jnp.int32)
    src_rows = jnp.concatenate([src, ar % N]).reshape(E_PAD // CHUNK, CHUNK)
    dst_rows = jnp.concatenate([dst, N + ar % (N_PAD - N)]).reshape(
        E_PAD // CHUNK, CHUNK)
    ones_col = jnp.ones((CHUNK, DEGW), jnp.float32)
    zero_col = jnp.zeros((RPT, DEGW), jnp.float32)
    zero_rows = jnp.zeros((RPT, D_IN), jnp.float32)

    degp = _sc_degree(dst_rows, ones_col, zero_col)
    dinv, xs = _tc_prescale(degp, x)
    p0 = _sc_propagate(xs, src_rows, dst_rows, zero_rows)
    hs2a, hs2b = _tc_layer1(p0, xs, dinv, W1, jnp.reshape(b1, (1, 512)), W2)
    p2a = _sc_propagate(hs2a, src_rows, dst_rows, zero_rows)
    p2b = _sc_propagate(hs2b, src_rows, dst_rows, zero_rows)
    hs3 = _tc_layer2(p2a, p2b, hs2a, hs2b, dinv, jnp.reshape(b2, (1, 256)), W3)
    p3 = _sc_propagate(hs3, src_rows, dst_rows, zero_rows)
    z3, S = _tc_layer3(p3, hs3, dinv, jnp.reshape(b3, (1, 128)))
    return _tc_tail(z3, S)


# speedup vs baseline: 19.8638x; 1.0018x over previous
import functools

import jax
import jax.numpy as jnp
from jax import lax
from jax.experimental import pallas as pl
from jax.experimental.pallas import tpu as pltpu
from jax.experimental.pallas import tpu_sc as plsc

N = 10000
E = 320000
D_IN = 128
NC, NS = 2, 16
NW = NC * NS
CHUNK = 128
CPT = 80
E_PAD = NW * CPT * CHUNK
N_PAD = 10240
RPT = N_PAD // NS
DEGW = 128

NBUF = 5
PBUF = 2
IB = 16
N_IBLK = CPT // IB
P_INNER = IB // PBUF

ROW_BLK = 400
N_BLKS = N // ROW_BLK

@functools.cache
def _sc_mesh():
    return plsc.VectorSubcoreMesh(
        core_axis_name="c", subcore_axis_name="s", num_cores=NC, num_subcores=NS
    )


def _sc_degree(dst_rows, ones_col, zero_col):
    @functools.partial(
        pl.kernel,
        out_type=jax.ShapeDtypeStruct((NC, N_PAD, DEGW), jnp.float32),
        mesh=_sc_mesh(),
        scratch_types=[
            pltpu.VMEM((CPT, CHUNK), jnp.int32),
            pltpu.VMEM((CHUNK, DEGW), jnp.float32),
            pltpu.VMEM_SHARED((N_PAD, DEGW), jnp.float32),
        ] + [pltpu.SemaphoreType.DMA] * NBUF,
    )
    def deg_kernel(dst_hbm, ones_hbm, zcol_hbm, out_hbm, dst_v, ones_v, acc,
                   *sems):
        c = lax.axis_index("c")
        s = lax.axis_index("s")
        w = c * NS + s
        pltpu.sync_copy(zcol_hbm, acc.at[pl.ds(s * RPT, RPT)])
        pltpu.sync_copy(ones_hbm, ones_v)
        pltpu.sync_copy(dst_hbm.at[pl.ds(w * CPT, CPT)], dst_v)
        plsc.subcore_barrier()

        def body(t, carry):
            base = t * NBUF
            descs = [
                pltpu.async_copy(ones_v, acc.at[dst_v.at[base + b]],
                                 sems[b], add=True)
                for b in range(NBUF)
            ]
            for d in descs:
                d.wait()
            return carry

        lax.fori_loop(0, CPT // NBUF, body, 0)
        plsc.subcore_barrier()
        pltpu.sync_copy(acc.at[pl.ds(s * RPT, RPT)],
                        out_hbm.at[c, pl.ds(s * RPT, RPT)])

    return deg_kernel(dst_rows, ones_col, zero_col)


def _sc_propagate(hs, src_rows, dst_rows, zero_rows):
    @functools.partial(
        pl.kernel,
        out_type=jax.ShapeDtypeStruct((NC, N_PAD, D_IN), jnp.float32),
        mesh=_sc_mesh(),
        scratch_types=[
            pltpu.VMEM((IB, CHUNK), jnp.int32),
            pltpu.VMEM((IB, CHUNK), jnp.int32),
        ] + [pltpu.VMEM((CHUNK, D_IN), jnp.float32)] * PBUF + [
            pltpu.VMEM_SHARED((N_PAD, D_IN), jnp.float32),
        ] + [pltpu.SemaphoreType.DMA] * (2 * PBUF),
    )
    def prop_kernel(hs_hbm, src_hbm, dst_hbm, zr_hbm, out_hbm,
                    src_v, dst_v, *rest):
        bufs = rest[:PBUF]
        acc = rest[PBUF]
        gsems = rest[PBUF + 1:PBUF + 1 + PBUF]
        ssems = rest[PBUF + 1 + PBUF:]
        c = lax.axis_index("c")
        s = lax.axis_index("s")
        w = c * NS + s
        pltpu.sync_copy(zr_hbm, acc.at[pl.ds(s * RPT, RPT)])
        plsc.subcore_barrier()

        def _drain_scatter(b):
            pltpu.make_async_copy(bufs[b], acc.at[dst_v.at[0]],
                                  ssems[b]).wait()

        def block(ib, carry):
            @pl.when(ib > 0)
            def _():
                for b in range(PBUF):
                    _drain_scatter(b)

            boff = w * CPT + ib * IB
            pltpu.sync_copy(src_hbm.at[pl.ds(boff, IB)], src_v)
            pltpu.sync_copy(dst_hbm.at[pl.ds(boff, IB)], dst_v)

            def body(t, carry2):
                base = t * PBUF

                @pl.when(t > 0)
                def _():
                    for b in range(PBUF):
                        _drain_scatter(b)

                gds = [
                    pltpu.async_copy(hs_hbm.at[src_v.at[base + b]], bufs[b],
                                     gsems[b], priority=1)
                    for b in range(PBUF)
                ]
                for b in range(PBUF):
                    gds[b].wait()
                    pltpu.async_copy(bufs[b], acc.at[dst_v.at[base + b]],
                                     ssems[b], add=True)
                return carry2

            lax.fori_loop(0, P_INNER, body, 0)
            return carry

        lax.fori_loop(0, N_IBLK, block, 0)
        for b in range(PBUF):
            _drain_scatter(b)
        plsc.subcore_barrier()
        pltpu.sync_copy(acc.at[pl.ds(s * RPT, RPT)],
                        out_hbm.at[c, pl.ds(s * RPT, RPT)])

    return prop_kernel(hs, src_rows, dst_rows, zero_rows)


def _tc_prescale(degp, x):
    def body(dp_ref, x_ref, dinv_ref, xs_ref):
        deg = dp_ref[0][:, :1] + dp_ref[1][:, :1] + 1.0
        dinv = lax.rsqrt(deg)
        dinv_ref[...] = dinv
        xs_ref[...] = x_ref[...] * dinv

    return pl.pallas_call(
        body,
        grid=(N_BLKS,),
        in_specs=[
            pl.BlockSpec((NC, ROW_BLK, DEGW), lambda i: (0, i, 0)),
            pl.BlockSpec((ROW_BLK, D_IN), lambda i: (i, 0)),
        ],
        out_specs=[
            pl.BlockSpec((ROW_BLK, 1), lambda i: (i, 0)),
            pl.BlockSpec((ROW_BLK, D_IN), lambda i: (i, 0)),
        ],
        out_shape=[
            jax.ShapeDtypeStruct((N, 1), jnp.float32),
            jax.ShapeDtypeStruct((N, D_IN), jnp.float32),
        ],
    )(degp, x)


def _tc_layer1(p0, xs, dinv, W1, b1, W2):
    def body(p_ref, xs_ref, dinv_ref, W1_ref, b1_ref, W2_ref, oa_ref, ob_ref):
        dinv = dinv_ref[...]
        p = (p_ref[0] + p_ref[1] + xs_ref[...]) * dinv
        z1 = jnp.maximum(
            jnp.dot(p, W1_ref[...], preferred_element_type=jnp.float32)
            + b1_ref[...], 0.0)
        h2 = jnp.dot(z1, W2_ref[...], preferred_element_type=jnp.float32)
        oa_ref[...] = h2[:, :128] * dinv
        ob_ref[...] = h2[:, 128:] * dinv

    return pl.pallas_call(
        body,
        grid=(N_BLKS,),
        in_specs=[
            pl.BlockSpec((NC, ROW_BLK, D_IN), lambda i: (0, i, 0)),
            pl.BlockSpec((ROW_BLK, D_IN), lambda i: (i, 0)),
            pl.BlockSpec((ROW_BLK, 1), lambda i: (i, 0)),
            pl.BlockSpec((D_IN, 512), lambda i: (0, 0)),
            pl.BlockSpec((1, 512), lambda i: (0, 0)),
            pl.BlockSpec((512, 256), lambda i: (0, 0)),
        ],
        out_specs=[
            pl.BlockSpec((ROW_BLK, 128), lambda i: (i, 0)),
            pl.BlockSpec((ROW_BLK, 128), lambda i: (i, 0)),
        ],
        out_shape=[
            jax.ShapeDtypeStruct((N, 128), jnp.float32),
            jax.ShapeDtypeStruct((N, 128), jnp.float32),
        ],
    )(p0, xs, dinv, W1, b1, W2)


def _tc_layer2(p2a, p2b, hs2a, hs2b, dinv, b2, W3):
    def body(pa_ref, pb_ref, ha_ref, hb_ref, dinv_ref, b2_ref, W3_ref, o_ref):
        dinv = dinv_ref[...]
        za = jnp.maximum(
            (pa_ref[0] + pa_ref[1] + ha_ref[...]) * dinv + b2_ref[:, :128], 0.0)
        zb = jnp.maximum(
            (pb_ref[0] + pb_ref[1] + hb_ref[...]) * dinv + b2_ref[:, 128:], 0.0)
        h3 = (jnp.dot(za, W3_ref[:128, :], preferred_element_type=jnp.float32)
              + jnp.dot(zb, W3_ref[128:, :], preferred_element_type=jnp.float32))
        o_ref[...] = h3 * dinv

    return pl.pallas_call(
        body,
        grid=(N_BLKS,),
        in_specs=[
            pl.BlockSpec((NC, ROW_BLK, 128), lambda i: (0, i, 0)),
            pl.BlockSpec((NC, ROW_BLK, 128), lambda i: (0, i, 0)),
            pl.BlockSpec((ROW_BLK, 128), lambda i: (i, 0)),
            pl.BlockSpec((ROW_BLK, 128), lambda i: (i, 0)),
            pl.BlockSpec((ROW_BLK, 1), lambda i: (i, 0)),
            pl.BlockSpec((1, 256), lambda i: (0, 0)),
            pl.BlockSpec((256, 128), lambda i: (0, 0)),
        ],
        out_specs=pl.BlockSpec((ROW_BLK, 128), lambda i: (i, 0)),
        out_shape=jax.ShapeDtypeStruct((N, 128), jnp.float32),
    )(p2a, p2b, hs2a, hs2b, dinv, b2, W3)


def _tc_layer3(p3, hs3, dinv, b3):
    def body(p_ref, h_ref, dinv_ref, b3_ref, z_ref, s_ref):
        z = jnp.maximum(
            (p_ref[0] + p_ref[1] + h_ref[...]) * dinv_ref[...] + b3_ref[...],
            0.0)
        z_ref[...] = z
        prev = jnp.where(pl.program_id(0) == 0, 0.0, s_ref[0, 0])
        s_ref[0, 0] = prev + jnp.sum(z)

    return pl.pallas_call(
        body,
        grid=(N_BLKS,),
        in_specs=[
            pl.BlockSpec((NC, ROW_BLK, 128), lambda i: (0, i, 0)),
            pl.BlockSpec((ROW_BLK, 128), lambda i: (i, 0)),
            pl.BlockSpec((ROW_BLK, 1), lambda i: (i, 0)),
            pl.BlockSpec((1, 128), lambda i: (0, 0)),
        ],
        out_specs=[
            pl.BlockSpec((ROW_BLK, 128), lambda i: (i, 0)),
            pl.BlockSpec(memory_space=pltpu.SMEM),
        ],
        out_shape=[
            jax.ShapeDtypeStruct((N, 128), jnp.float32),
            jax.ShapeDtypeStruct((1, 1), jnp.float32),
        ],
    )(p3, hs3, dinv, b3)


def _tc_tail(z3, S):
    def body(z_ref, s_ref, o_ref):
        z = z_ref[...] / s_ref[0, 0]
        t = jnp.tanh(z)
        t = t * t
        rn = jnp.sqrt(jnp.sum(t * t, axis=1, keepdims=True))
        o_ref[...] = t / jnp.maximum(rn, 1e-12)

    return pl.pallas_call(
        body,
        grid=(N_BLKS,),
        in_specs=[
            pl.BlockSpec((ROW_BLK, 128), lambda i: (i, 0)),
            pl.BlockSpec(memory_space=pltpu.SMEM),
        ],
        out_specs=pl.BlockSpec((ROW_BLK, 128), lambda i: (i, 0)),
        out_shape=jax.ShapeDtypeStruct((N, 128), jnp.float32),
    )(z3, S)


def kernel(x, edge_index, W1, b1, W2, b2, W3, b3):
    src = edge_index[0].astype(jnp.int32)
    dst = edge_index[1].astype(jnp.int32)
    padn = E_PAD - E
    ar = jnp.arange(padn, dtype=jnp.int32)
    src_rows = jnp.concatenate([src, ar % N]).reshape(E_PAD // CHUNK, CHUNK)
    dst_rows = jnp.concatenate([dst, N + ar % (N_PAD - N)]).reshape(
        E_PAD // CHUNK, CHUNK)
    ones_col = jnp.ones((CHUNK, DEGW), jnp.float32)
    zero_col = jnp.zeros((RPT, DEGW), jnp.float32)
    zero_rows = jnp.zeros((RPT, D_IN), jnp.float32)

    degp = _sc_degree(dst_rows, ones_col, zero_col)
    dinv, xs = _tc_prescale(degp, x)
    p0 = _sc_propagate(xs, src_rows, dst_rows, zero_rows)
    hs2a, hs2b = _tc_layer1(p0, xs, dinv, W1, jnp.reshape(b1, (1, 512)), W2)
    p2a = _sc_propagate(hs2a, src_rows, dst_rows, zero_rows)
    p2b = _sc_propagate(hs2b, src_rows, dst_rows, zero_rows)
    hs3 = _tc_layer2(p2a, p2b, hs2a, hs2b, dinv, jnp.reshape(b2, (1, 256)), W3)
    p3 = _sc_propagate(hs3, src_rows, dst_rows, zero_rows)
    z3, S = _tc_layer3(p3, hs3, dinv, jnp.reshape(b3, (1, 128)))
    return _tc_tail(z3, S)


# speedup vs baseline: 20.0493x; 1.0093x over previous
import functools

import jax
import jax.numpy as jnp
from jax import lax
from jax.experimental import pallas as pl
from jax.experimental.pallas import tpu as pltpu
from jax.experimental.pallas import tpu_sc as plsc

N = 10000
E = 320000
D_IN = 128
NC, NS = 2, 16
NW = NC * NS
CHUNK = 128
CPT = 80
E_PAD = NW * CPT * CHUNK
N_PAD = 10240
RPT = N_PAD // NS
DEGW = 128

NBUF = 5
PBUF = 2
IB = 16
N_IBLK = CPT // IB
P_INNER = IB // PBUF

ROW_BLK = 400
N_BLKS = N // ROW_BLK

@functools.cache
def _sc_mesh():
    return plsc.VectorSubcoreMesh(
        core_axis_name="c", subcore_axis_name="s", num_cores=NC, num_subcores=NS
    )


def _sc_degree(dst_rows, ones_col, zero_col):
    @functools.partial(
        pl.kernel,
        out_type=jax.ShapeDtypeStruct((NC, N_PAD, DEGW), jnp.float32),
        mesh=_sc_mesh(),
        scratch_types=[
            pltpu.VMEM((CPT, CHUNK), jnp.int32),
            pltpu.VMEM((CHUNK, DEGW), jnp.float32),
            pltpu.VMEM_SHARED((N_PAD, DEGW), jnp.float32),
        ] + [pltpu.SemaphoreType.DMA] * NBUF,
    )
    def deg_kernel(dst_hbm, ones_hbm, zcol_hbm, out_hbm, dst_v, ones_v, acc,
                   *sems):
        c = lax.axis_index("c")
        s = lax.axis_index("s")
        w = c * NS + s
        pltpu.sync_copy(zcol_hbm, acc.at[pl.ds(s * RPT, RPT)])
        pltpu.sync_copy(ones_hbm, ones_v)
        pltpu.sync_copy(dst_hbm.at[pl.ds(w * CPT, CPT)], dst_v)
        plsc.subcore_barrier()

        def body(t, carry):
            base = t * NBUF
            descs = [
                pltpu.async_copy(ones_v, acc.at[dst_v.at[base + b]],
                                 sems[b], add=True)
                for b in range(NBUF)
            ]
            for d in descs:
                d.wait()
            return carry

        lax.fori_loop(0, CPT // NBUF, body, 0)
        plsc.subcore_barrier()
        pltpu.sync_copy(acc.at[pl.ds(s * RPT, RPT)],
                        out_hbm.at[c, pl.ds(s * RPT, RPT)])

    return deg_kernel(dst_rows, ones_col, zero_col)


def _sc_propagate(hs, src_rows, dst_rows, zero_rows):
    @functools.partial(
        pl.kernel,
        out_type=jax.ShapeDtypeStruct((NC, N_PAD, D_IN), jnp.float32),
        mesh=_sc_mesh(),
        scratch_types=[
            pltpu.VMEM((IB, CHUNK), jnp.int32),
            pltpu.VMEM((IB, CHUNK), jnp.int32),
        ] + [pltpu.VMEM((CHUNK, D_IN), jnp.float32)] * PBUF + [
            pltpu.VMEM_SHARED((N_PAD, D_IN), jnp.float32),
        ] + [pltpu.SemaphoreType.DMA] * (2 * PBUF),
    )
    def prop_kernel(hs_hbm, src_hbm, dst_hbm, zr_hbm, out_hbm,
                    src_v, dst_v, *rest):
        bufs = rest[:PBUF]
        acc = rest[PBUF]
        gsems = rest[PBUF + 1:PBUF + 1 + PBUF]
        ssems = rest[PBUF + 1 + PBUF:]
        _propagate_body(hs_hbm, src_hbm, dst_hbm, zr_hbm, out_hbm,
                        src_v, dst_v, bufs, acc, gsems, ssems)

    return prop_kernel(hs, src_rows, dst_rows, zero_rows)


def _propagate_body(hs_hbm, src_hbm, dst_hbm, zr_hbm, out_hbm,
                    src_v, dst_v, bufs, acc, gsems, ssems):
        c = lax.axis_index("c")
        s = lax.axis_index("s")
        w = c * NS + s
        pltpu.sync_copy(zr_hbm, acc.at[pl.ds(s * RPT, RPT)])
        plsc.subcore_barrier()

        def _drain_scatter(b):
            pltpu.make_async_copy(bufs[b], acc.at[dst_v.at[0]],
                                  ssems[b]).wait()

        def block(ib, carry):
            @pl.when(ib > 0)
            def _():
                for b in range(PBUF):
                    _drain_scatter(b)

            boff = w * CPT + ib * IB
            pltpu.sync_copy(src_hbm.at[pl.ds(boff, IB)], src_v)
            pltpu.sync_copy(dst_hbm.at[pl.ds(boff, IB)], dst_v)

            def body(t, carry2):
                base = t * PBUF

                @pl.when(t > 0)
                def _():
                    for b in range(PBUF):
                        _drain_scatter(b)

                gds = [
                    pltpu.async_copy(hs_hbm.at[src_v.at[base + b]], bufs[b],
                                     gsems[b], priority=1)
                    for b in range(PBUF)
                ]
                for b in range(PBUF):
                    gds[b].wait()
                    pltpu.async_copy(bufs[b], acc.at[dst_v.at[base + b]],
                                     ssems[b], add=True)
                return carry2

            lax.fori_loop(0, P_INNER, body, 0)
            return carry

        lax.fori_loop(0, N_IBLK, block, 0)
        for b in range(PBUF):
            _drain_scatter(b)
        plsc.subcore_barrier()
        pltpu.sync_copy(acc.at[pl.ds(s * RPT, RPT)],
                        out_hbm.at[c, pl.ds(s * RPT, RPT)])


def _sc_propagate2(hsa, hsb, src_rows, dst_rows, zero_rows):
    @functools.partial(
        pl.kernel,
        out_type=jax.ShapeDtypeStruct((2, NC, N_PAD, D_IN), jnp.float32),
        mesh=_sc_mesh(),
        scratch_types=[
            pltpu.VMEM((IB, CHUNK), jnp.int32),
            pltpu.VMEM((IB, CHUNK), jnp.int32),
        ] + [pltpu.VMEM((CHUNK, D_IN), jnp.float32)] * PBUF + [
            pltpu.VMEM_SHARED((N_PAD, D_IN), jnp.float32),
        ] + [pltpu.SemaphoreType.DMA] * (2 * PBUF),
    )
    def prop2_kernel(hsa_hbm, hsb_hbm, src_hbm, dst_hbm, zr_hbm, out_hbm,
                     src_v, dst_v, *rest):
        bufs = rest[:PBUF]
        acc = rest[PBUF]
        gsems = rest[PBUF + 1:PBUF + 1 + PBUF]
        ssems = rest[PBUF + 1 + PBUF:]
        _propagate_body(hsa_hbm, src_hbm, dst_hbm, zr_hbm, out_hbm.at[0],
                        src_v, dst_v, bufs, acc, gsems, ssems)
        _propagate_body(hsb_hbm, src_hbm, dst_hbm, zr_hbm, out_hbm.at[1],
                        src_v, dst_v, bufs, acc, gsems, ssems)

    return prop2_kernel(hsa, hsb, src_rows, dst_rows, zero_rows)


def _tc_prescale(degp, x):
    def body(dp_ref, x_ref, dinv_ref, xs_ref):
        deg = dp_ref[0][:, :1] + dp_ref[1][:, :1] + 1.0
        dinv = lax.rsqrt(deg)
        dinv_ref[...] = dinv
        xs_ref[...] = x_ref[...] * dinv

    return pl.pallas_call(
        body,
        grid=(N_BLKS,),
        in_specs=[
            pl.BlockSpec((NC, ROW_BLK, DEGW), lambda i: (0, i, 0)),
            pl.BlockSpec((ROW_BLK, D_IN), lambda i: (i, 0)),
        ],
        out_specs=[
            pl.BlockSpec((ROW_BLK, 1), lambda i: (i, 0)),
            pl.BlockSpec((ROW_BLK, D_IN), lambda i: (i, 0)),
        ],
        out_shape=[
            jax.ShapeDtypeStruct((N, 1), jnp.float32),
            jax.ShapeDtypeStruct((N, D_IN), jnp.float32),
        ],
    )(degp, x)


def _tc_layer1(p0, xs, dinv, W1, b1, W2):
    def body(p_ref, xs_ref, dinv_ref, W1_ref, b1_ref, W2_ref, oa_ref, ob_ref):
        dinv = dinv_ref[...]
        p = (p_ref[0] + p_ref[1] + xs_ref[...]) * dinv
        z1 = jnp.maximum(
            jnp.dot(p, W1_ref[...], preferred_element_type=jnp.float32)
            + b1_ref[...], 0.0)
        h2 = jnp.dot(z1, W2_ref[...], preferred_element_type=jnp.float32)
        oa_ref[...] = h2[:, :128] * dinv
        ob_ref[...] = h2[:, 128:] * dinv

    return pl.pallas_call(
        body,
        grid=(N_BLKS,),
        in_specs=[
            pl.BlockSpec((NC, ROW_BLK, D_IN), lambda i: (0, i, 0)),
            pl.BlockSpec((ROW_BLK, D_IN), lambda i: (i, 0)),
            pl.BlockSpec((ROW_BLK, 1), lambda i: (i, 0)),
            pl.BlockSpec((D_IN, 512), lambda i: (0, 0)),
            pl.BlockSpec((1, 512), lambda i: (0, 0)),
            pl.BlockSpec((512, 256), lambda i: (0, 0)),
        ],
        out_specs=[
            pl.BlockSpec((ROW_BLK, 128), lambda i: (i, 0)),
            pl.BlockSpec((ROW_BLK, 128), lambda i: (i, 0)),
        ],
        out_shape=[
            jax.ShapeDtypeStruct((N, 128), jnp.float32),
            jax.ShapeDtypeStruct((N, 128), jnp.float32),
        ],
    )(p0, xs, dinv, W1, b1, W2)


def _tc_layer2(p2, hs2a, hs2b, dinv, b2, W3):
    def body(p_ref, ha_ref, hb_ref, dinv_ref, b2_ref, W3_ref, o_ref):
        dinv = dinv_ref[...]
        za = jnp.maximum(
            (p_ref[0, 0] + p_ref[0, 1] + ha_ref[...]) * dinv + b2_ref[:, :128],
            0.0)
        zb = jnp.maximum(
            (p_ref[1, 0] + p_ref[1, 1] + hb_ref[...]) * dinv + b2_ref[:, 128:],
            0.0)
        h3 = (jnp.dot(za, W3_ref[:128, :], preferred_element_type=jnp.float32)
              + jnp.dot(zb, W3_ref[128:, :], preferred_element_type=jnp.float32))
        o_ref[...] = h3 * dinv

    return pl.pallas_call(
        body,
        grid=(N_BLKS,),
        in_specs=[
            pl.BlockSpec((2, NC, ROW_BLK, 128), lambda i: (0, 0, i, 0)),
            pl.BlockSpec((ROW_BLK, 128), lambda i: (i, 0)),
            pl.BlockSpec((ROW_BLK, 128), lambda i: (i, 0)),
            pl.BlockSpec((ROW_BLK, 1), lambda i: (i, 0)),
            pl.BlockSpec((1, 256), lambda i: (0, 0)),
            pl.BlockSpec((256, 128), lambda i: (0, 0)),
        ],
        out_specs=pl.BlockSpec((ROW_BLK, 128), lambda i: (i, 0)),
        out_shape=jax.ShapeDtypeStruct((N, 128), jnp.float32),
    )(p2, hs2a, hs2b, dinv, b2, W3)


def _tc_layer3(p3, hs3, dinv, b3):
    def body(p_ref, h_ref, dinv_ref, b3_ref, z_ref, s_ref):
        z = jnp.maximum(
            (p_ref[0] + p_ref[1] + h_ref[...]) * dinv_ref[...] + b3_ref[...],
            0.0)
        z_ref[...] = z
        prev = jnp.where(pl.program_id(0) == 0, 0.0, s_ref[0, 0])
        s_ref[0, 0] = prev + jnp.sum(z)

    return pl.pallas_call(
        body,
        grid=(N_BLKS,),
        in_specs=[
            pl.BlockSpec((NC, ROW_BLK, 128), lambda i: (0, i, 0)),
            pl.BlockSpec((ROW_BLK, 128), lambda i: (i, 0)),
            pl.BlockSpec((ROW_BLK, 1), lambda i: (i, 0)),
            pl.BlockSpec((1, 128), lambda i: (0, 0)),
        ],
        out_specs=[
            pl.BlockSpec((ROW_BLK, 128), lambda i: (i, 0)),
            pl.BlockSpec(memory_space=pltpu.SMEM),
        ],
        out_shape=[
            jax.ShapeDtypeStruct((N, 128), jnp.float32),
            jax.ShapeDtypeStruct((1, 1), jnp.float32),
        ],
    )(p3, hs3, dinv, b3)


def _tc_tail(z3, S):
    def body(z_ref, s_ref, o_ref):
        z = z_ref[...] / s_ref[0, 0]
        t = jnp.tanh(z)
        t = t * t
        rn = jnp.sqrt(jnp.sum(t * t, axis=1, keepdims=True))
        o_ref[...] = t / jnp.maximum(rn, 1e-12)

    return pl.pallas_call(
        body,
        grid=(N_BLKS,),
        in_specs=[
            pl.BlockSpec((ROW_BLK, 128), lambda i: (i, 0)),
            pl.BlockSpec(memory_space=pltpu.SMEM),
        ],
        out_specs=pl.BlockSpec((ROW_BLK, 128), lambda i: (i, 0)),
        out_shape=jax.ShapeDtypeStruct((N, 128), jnp.float32),
    )(z3, S)


def kernel(x, edge_index, W1, b1, W2, b2, W3, b3):
    src = edge_index[0].astype(jnp.int32)
    dst = edge_index[1].astype(jnp.int32)
    padn = E_PAD - E
    ar = jnp.arange(padn, dtype=jnp.int32)
    src_rows = jnp.concatenate([src, ar % N]).reshape(E_PAD // CHUNK, CHUNK)
    dst_rows = jnp.concatenate([dst, N + ar % (N_PAD - N)]).reshape(
        E_PAD // CHUNK, CHUNK)
    ones_col = jnp.ones((CHUNK, DEGW), jnp.float32)
    zero_col = jnp.zeros((RPT, DEGW), jnp.float32)
    zero_rows = jnp.zeros((RPT, D_IN), jnp.float32)

    degp = _sc_degree(dst_rows, ones_col, zero_col)
    dinv, xs = _tc_prescale(degp, x)
    p0 = _sc_propagate(xs, src_rows, dst_rows, zero_rows)
    hs2a, hs2b = _tc_layer1(p0, xs, dinv, W1, jnp.reshape(b1, (1, 512)), W2)
    p2 = _sc_propagate2(hs2a, hs2b, src_rows, dst_rows, zero_rows)
    hs3 = _tc_layer2(p2, hs2a, hs2b, dinv, jnp.reshape(b2, (1, 256)), W3)
    p3 = _sc_propagate(hs3, src_rows, dst_rows, zero_rows)
    z3, S = _tc_layer3(p3, hs3, dinv, jnp.reshape(b3, (1, 128)))
    return _tc_tail(z3, S)
